# Initial kernel scaffold; baseline (speedup 1.0000x reference)
import jax
import jax.numpy as jnp
from jax.experimental import pallas as pl


def kernel(device_node_id, ip_node_id, user_node_id, note_node_id, event_node_id, device_cat, event_cat, user_dense, node_type_table, dev_emb0, dev_emb1, dev_emb2, W_dev, ev_emb0, ev_emb1, W_ev, W_user, b_user):
    raise NotImplementedError("write your pallas kernel here")


# speedup vs baseline: 8.7276x; 8.7276x over previous
import functools

import jax
import jax.numpy as jnp
from jax import lax
from jax.experimental import pallas as pl
from jax.experimental.pallas import tpu as pltpu
from jax.experimental.pallas import tpu_sc as plsc

HID = 64
B = 128
NW = 32

N_DEV = 100000
N_IP = 100000
N_USER = 100000
N_NOTE = 150000
N_EV = 50000
N_TOTAL = 500000


def _cdiv(a, b):
    return (a + b - 1) // b


def _pad8(n):
    return _cdiv(n, 8) * 8


C_DEV = _pad8(_cdiv(N_DEV, B))
C_IP = _pad8(_cdiv(N_IP, B))
C_USER = _pad8(_cdiv(N_USER, B))
C_NOTE = _pad8(_cdiv(N_NOTE, B))
C_EV = _pad8(_cdiv(N_EV, B))

CPW_DEV = _cdiv(C_DEV, NW)
CPW_IP = _cdiv(C_IP, NW)
CPW_USER = _cdiv(C_USER, NW)
CPW_NOTE = _cdiv(C_NOTE, NW)
CPW_EV = _cdiv(C_EV, NW)


def _stage_rows(cpw):
    return (_cdiv(cpw, 8) + 1) * 8


S_DEV = _stage_rows(CPW_DEV)
S_IP = _stage_rows(CPW_IP)
S_USER = _stage_rows(CPW_USER)
S_NOTE = _stage_rows(CPW_NOTE)
S_EV = _stage_rows(CPW_EV)


def _tables_dev_body(de0, de1, de2, wdev, ntt, t01, t2, ipb, noteb):
    w = wdev[...]
    n = ntt[...]
    t0p = jnp.dot(de0[...], w[0:16, :], preferred_element_type=jnp.float32)
    t0p = t0p + n[0:1, :]
    t1 = jnp.dot(de1[...], w[16:24, :], preferred_element_type=jnp.float32)
    t01[...] = t0p[:, None, :] + t1[None, :, :]
    t2[...] = jnp.dot(de2[...], w[24:40, :], preferred_element_type=jnp.float32)
    ipb[...] = jnp.broadcast_to(n[1:2, :], (B, HID))
    noteb[...] = jnp.broadcast_to(n[3:4, :], (B, HID))


def _tables_ev_body(ee0, ee1, wev, ntt, te01):
    w = wev[...]
    te0 = jnp.dot(ee0[...], w[0:16, :], preferred_element_type=jnp.float32)
    te0 = te0 + ntt[...][4:5, :]
    te1 = jnp.dot(ee1[...], w[16:24, :], preferred_element_type=jnp.float32)
    te01[...] = te0[:, None, :] + te1[None, :, :]


def _user_body(ud, wu, bu, o):
    o[...] = jnp.dot(ud[...], wu[...], preferred_element_type=jnp.float32) + bu[...]


def _tc_tables_dev(de0, de1, de2, wdev, ntt):
    return pl.pallas_call(
        _tables_dev_body,
        out_shape=[
            jax.ShapeDtypeStruct((1000, 50, HID), jnp.float32),
            jax.ShapeDtypeStruct((20000, HID), jnp.float32),
            jax.ShapeDtypeStruct((B, HID), jnp.float32),
            jax.ShapeDtypeStruct((B, HID), jnp.float32),
        ],
    )(de0, de1, de2, wdev, ntt)


def _tc_tables_ev(ee0, ee1, wev, ntt):
    return pl.pallas_call(
        _tables_ev_body,
        out_shape=jax.ShapeDtypeStruct((500, 100, HID), jnp.float32),
    )(ee0, ee1, wev, ntt)


def _tc_user(ud, wu, bu):
    nrows = C_USER * B
    return pl.pallas_call(
        _user_body,
        grid=(49,),
        in_specs=[
            pl.BlockSpec((2048, 2), lambda i: (i, 0)),
            pl.BlockSpec((2, HID), lambda i: (0, 0)),
            pl.BlockSpec((1, HID), lambda i: (0, 0)),
        ],
        out_specs=pl.BlockSpec((2048, HID), lambda i: (i, 0)),
        out_shape=jax.ShapeDtypeStruct((nrows, HID), jnp.float32),
    )(ud, wu, bu)


_MESH = plsc.VectorSubcoreMesh(
    core_axis_name="c", subcore_axis_name="s", num_cores=2, num_subcores=16)


def _sc_body(d_nid, d_i0, d_i1, d_i2, i_nid, u_nid, n_nid, e_nid, e_i0, e_i1,
             t01, t2, te01, xu, ipb, noteb, out,
             st_nid, st_a, st_b, acc, bb, cbuf, gs0, gs1, ss):
    wid = lax.axis_index("s") * 2 + lax.axis_index("c")

    def ranges(cpw, s, nc, rot):
        ws = lax.rem(wid + rot, NW)
        c_lo = ws * cpw
        cl = jnp.minimum((c_lo // 8) * 8, nc - s)
        return c_lo, pl.multiple_of(cl, 8)

    def combine(rows, mult):
        def body(r, _):
            for k in range(8):
                sl = pl.ds(k * 16, 16)
                st_a[r, sl] = st_a[r, sl] * mult + st_b[r, sl]
            return 0
        lax.fori_loop(0, rows, body, 0)

    c_lo, cl = ranges(CPW_DEV, S_DEV, C_DEV, 0)
    pltpu.sync_copy(d_nid.at[pl.ds(cl, S_DEV)], st_nid.at[pl.ds(0, S_DEV)])
    pltpu.sync_copy(d_i0.at[pl.ds(cl, S_DEV)], st_a)
    pltpu.sync_copy(d_i1.at[pl.ds(cl, S_DEV)], st_b)
    combine(S_DEV, 50)
    pltpu.sync_copy(d_i2.at[pl.ds(cl, S_DEV)], st_b)

    def dev_chunk(t, _):
        c = c_lo + t

        @pl.when(c < C_DEV)
        def _():
            r = c - cl
            g0 = pltpu.async_copy(t01.at[st_a.at[r]], acc, gs0)
            g1 = pltpu.async_copy(t2.at[st_b.at[r]], bb, gs1)
            g0.wait()
            g1.wait()

            def addrow(j, _2):
                for k in range(4):
                    sl = pl.ds(k * 16, 16)
                    plsc.addupdate(acc.at[j, sl], bb[j, sl])
                return 0
            lax.fori_loop(0, B, addrow, 0)
            pltpu.async_copy(acc, out.at[st_nid.at[r]], ss).wait()
        return 0
    lax.fori_loop(0, CPW_DEV, dev_chunk, 0)

    c_lo, cl = ranges(CPW_IP, S_IP, C_IP, 7)
    pltpu.sync_copy(ipb, cbuf)
    pltpu.sync_copy(i_nid.at[pl.ds(cl, S_IP)], st_nid.at[pl.ds(0, S_IP)])

    def ip_chunk(t, _):
        c = c_lo + t

        @pl.when(c < C_IP)
        def _():
            r = c - cl
            pltpu.async_copy(cbuf, out.at[st_nid.at[r]], ss).wait()
        return 0
    lax.fori_loop(0, CPW_IP, ip_chunk, 0)

    c_lo, cl = ranges(CPW_USER, S_USER, C_USER, 13)
    pltpu.sync_copy(u_nid.at[pl.ds(cl, S_USER)], st_nid.at[pl.ds(0, S_USER)])

    def user_chunk(t, _):
        c = c_lo + t

        @pl.when(c < C_USER)
        def _():
            r = c - cl
            pltpu.sync_copy(xu.at[pl.ds(c * B, B)], acc)
            pltpu.async_copy(acc, out.at[st_nid.at[r]], ss).wait()
        return 0
    lax.fori_loop(0, CPW_USER, user_chunk, 0)

    c_lo, cl = ranges(CPW_NOTE, S_NOTE, C_NOTE, 19)
    pltpu.sync_copy(noteb, cbuf)
    pltpu.sync_copy(n_nid.at[pl.ds(cl, S_NOTE)], st_nid)

    def note_chunk(t, _):
        c = c_lo + t

        @pl.when(c < C_NOTE)
        def _():
            r = c - cl
            pltpu.async_copy(cbuf, out.at[st_nid.at[r]], ss).wait()
        return 0
    lax.fori_loop(0, CPW_NOTE, note_chunk, 0)

    c_lo, cl = ranges(CPW_EV, S_EV, C_EV, 26)
    pltpu.sync_copy(e_nid.at[pl.ds(cl, S_EV)], st_nid.at[pl.ds(0, S_EV)])
    pltpu.sync_copy(e_i0.at[pl.ds(cl, S_EV)], st_a.at[pl.ds(0, S_EV)])
    pltpu.sync_copy(e_i1.at[pl.ds(cl, S_EV)], st_b.at[pl.ds(0, S_EV)])
    combine(S_EV, 100)

    def ev_chunk(t, _):
        c = c_lo + t

        @pl.when(c < C_EV)
        def _():
            r = c - cl
            pltpu.async_copy(te01.at[st_a.at[r]], acc, gs0).wait()
            pltpu.async_copy(acc, out.at[st_nid.at[r]], ss).wait()
        return 0
    lax.fori_loop(0, CPW_EV, ev_chunk, 0)


_sc_scatter = functools.partial(
    pl.kernel,
    out_type=jax.ShapeDtypeStruct((N_TOTAL, HID), jnp.float32),
    mesh=_MESH,
    scratch_types=[
        pltpu.VMEM((S_NOTE, B), jnp.int32),
        pltpu.VMEM((S_DEV, B), jnp.int32),
        pltpu.VMEM((S_DEV, B), jnp.int32),
        pltpu.VMEM((B, HID), jnp.float32),
        pltpu.VMEM((B, HID), jnp.float32),
        pltpu.VMEM((B, HID), jnp.float32),
        pltpu.SemaphoreType.DMA,
        pltpu.SemaphoreType.DMA,
        pltpu.SemaphoreType.DMA,
    ],
    compiler_params=pltpu.CompilerParams(use_tc_tiling_on_sc=False),
)(_sc_body)


def _pad_chunks(x, nc):
    return jnp.pad(x, (0, nc * B - x.shape[0]), mode='edge').reshape(nc, B)


def kernel(device_node_id, ip_node_id, user_node_id, note_node_id,
           event_node_id, device_cat, event_cat, user_dense,
           node_type_table, dev_emb0, dev_emb1, dev_emb2, W_dev,
           ev_emb0, ev_emb1, W_ev, W_user, b_user):
    i32 = jnp.int32
    d_nid = _pad_chunks(device_node_id.astype(i32), C_DEV)
    d_i0 = _pad_chunks(device_cat[:, 0].astype(i32), C_DEV)
    d_i1 = _pad_chunks(device_cat[:, 1].astype(i32), C_DEV)
    d_i2 = _pad_chunks(device_cat[:, 2].astype(i32), C_DEV)
    i_nid = _pad_chunks(ip_node_id.astype(i32), C_IP)
    u_nid = _pad_chunks(user_node_id.astype(i32), C_USER)
    n_nid = _pad_chunks(note_node_id.astype(i32), C_NOTE)
    e_nid = _pad_chunks(event_node_id.astype(i32), C_EV)
    e_i0 = _pad_chunks(event_cat[:, 0].astype(i32), C_EV)
    e_i1 = _pad_chunks(event_cat[:, 1].astype(i32), C_EV)

    t01_3, t2, ipb, noteb = _tc_tables_dev(
        dev_emb0, dev_emb1, dev_emb2, W_dev, node_type_table)
    te01_3 = _tc_tables_ev(ev_emb0, ev_emb1, W_ev, node_type_table)
    t01 = t01_3.reshape(1000 * 50, HID)
    te01 = te01_3.reshape(500 * 100, HID)

    ud = jnp.pad(user_dense, ((0, C_USER * B - N_USER), (0, 0)), mode='edge')
    xu = _tc_user(ud, W_user, b_user.reshape(1, HID))

    return _sc_scatter(d_nid, d_i0, d_i1, d_i2, i_nid, u_nid, n_nid,
                       e_nid, e_i0, e_i1, t01, t2, te01, xu, ipb, noteb)
